# baseline (speedup 1.0000x reference)
import functools

import jax
import jax.numpy as jnp
from jax import lax
from jax.experimental import pallas as pl
from jax.experimental.pallas import tpu as pltpu
from jax.experimental.pallas import tpu_sc as plsc

_INPUT_DIM = 1024
_HIDDEN = 512
_ENC_DIM = 256
_K = 8192
_NBLK = 6
_B = 4096
_BT = 1024
_BTD = 2048
_KC = 8192
_NW = 32
_BPW = _B // _NW

_f32 = jnp.float32


def _ln(x, g, b):
    m = jnp.mean(x, axis=-1, keepdims=True)
    v = jnp.var(x, axis=-1, keepdims=True)
    return (x - m) / jnp.sqrt(v + 1e-5) * g + b


def _enc_mlp(p, x):
    h = _ln(x @ p['W0'] + p['b0'], p['g0'], p['be0'])
    for blk in p['blocks']:
        t = jnp.tanh(_ln(h @ blk['W'] + blk['b'], blk['g'], blk['be']))
        t = _ln(t @ blk['W'] + blk['b'], blk['g'], blk['be'])
        h = jnp.tanh(t + h)
    return h @ p['Wf'] + p['bf']


def _quant_body(zf_ref, zn_ref, en_ref, e_ref, idx_ref):
    zf = zf_ref[...]
    zn = zn_ref[...]
    best_d = jnp.full((_BT, 1), 3.0e38, _f32)
    best_i = jnp.zeros((_BT, 1), jnp.int32)
    for c in range(_K // _KC):
        ec = e_ref[c * _KC:(c + 1) * _KC, :]
        en = en_ref[:, c * _KC:(c + 1) * _KC]
        s = lax.dot_general(zf, ec, (((1,), (1,)), ((), ())),
                            preferred_element_type=_f32)
        d = 1.0 - s / (zn * en)
        cm = jnp.min(d, axis=1, keepdims=True)
        io = lax.broadcasted_iota(jnp.int32, (_BT, _KC), 1)
        ci = jnp.min(jnp.where(d == cm, io, _K), axis=1, keepdims=True) + c * _KC
        upd = cm < best_d
        best_i = jnp.where(upd, ci, best_i)
        best_d = jnp.where(upd, cm, best_d)
    idx_ref[...] = jnp.broadcast_to(best_i, (_BT, 128))


def _run_quant(z, zn, en, emb):
    def im_b(i):
        return (i, 0)

    def im_w(i):
        return (0, 0)

    idx2d = pl.pallas_call(
        _quant_body,
        grid=(_B // _BT,),
        in_specs=[
            pl.BlockSpec((_BT, _ENC_DIM), im_b),
            pl.BlockSpec((_BT, 1), im_b),
            pl.BlockSpec((1, _K), im_w),
            pl.BlockSpec((_K, _ENC_DIM), im_w),
        ],
        out_specs=pl.BlockSpec((_BT, 128), im_b),
        out_shape=jax.ShapeDtypeStruct((_B, 128), jnp.int32),
    )(z, zn, en, emb)
    return idx2d[:, 0]


def _sc_gather(emb, idx):
    mesh = plsc.VectorSubcoreMesh(core_axis_name="c", subcore_axis_name="s")

    @functools.partial(
        pl.kernel,
        mesh=mesh,
        out_type=jax.ShapeDtypeStruct((_B, _ENC_DIM), _f32),
        scratch_types=[
            pltpu.VMEM((_BPW,), jnp.int32),
            pltpu.VMEM((_BPW, _ENC_DIM), _f32),
            pltpu.SemaphoreType.DMA,
        ],
    )
    def gk(table_hbm, idx_hbm, out_hbm, idx_v, rows_v, sem):
        wid = lax.axis_index("s") * 2 + lax.axis_index("c")
        base = wid * _BPW
        pltpu.sync_copy(idx_hbm.at[pl.ds(base, _BPW)], idx_v)
        pltpu.async_copy(table_hbm.at[idx_v], rows_v, sem).wait()
        pltpu.sync_copy(rows_v, out_hbm.at[pl.ds(base, _BPW)])

    return gk(emb, idx)


def _ln_in(h, g, b):
    m = jnp.mean(h, axis=-1, keepdims=True)
    v = jnp.mean((h - m) ** 2, axis=-1, keepdims=True)
    return (h - m) / jnp.sqrt(v + 1e-5) * g + b


def _dec_body(zq_ref, z_ref, w0_ref, v0_ref, wb0, wb1, wb2, wb3, wb4, wb5,
              bb_ref, gb_ref, beb_ref, wf_ref, bf_ref, out_ref):
    zz = z_ref[...]
    h_in = zz + (zq_ref[...] - zz)
    h = jnp.dot(h_in, w0_ref[...], preferred_element_type=_f32)
    h = _ln_in(h + v0_ref[0:1, :], v0_ref[1:2, :], v0_ref[2:3, :])
    w_refs = (wb0, wb1, wb2, wb3, wb4, wb5)
    for i in range(_NBLK):
        w = w_refs[i][...]
        b = bb_ref[i:i + 1, :]
        g = gb_ref[i:i + 1, :]
        be = beb_ref[i:i + 1, :]
        t = jnp.tanh(_ln_in(jnp.dot(h, w, preferred_element_type=_f32) + b, g, be))
        t = _ln_in(jnp.dot(t, w, preferred_element_type=_f32) + b, g, be)
        h = jnp.tanh(t + h)
    out_ref[...] = jnp.dot(h, wf_ref[...], preferred_element_type=_f32) + bf_ref[...]


def _run_dec(z_q, z, dec):
    wb = [blk['W'] for blk in dec['blocks']]
    bb = jnp.stack([blk['b'] for blk in dec['blocks']])
    gb = jnp.stack([blk['g'] for blk in dec['blocks']])
    beb = jnp.stack([blk['be'] for blk in dec['blocks']])
    v0 = jnp.stack([dec['b0'], dec['g0'], dec['be0']])
    bf = dec['bf'].reshape(1, _INPUT_DIM)

    def im_b(i):
        return (i, 0)

    def im_w(i):
        return (0, 0)

    in_specs = [
        pl.BlockSpec((_BTD, _ENC_DIM), im_b),
        pl.BlockSpec((_BTD, _ENC_DIM), im_b),
        pl.BlockSpec((_ENC_DIM, _HIDDEN), im_w),
        pl.BlockSpec((3, _HIDDEN), im_w),
        *[pl.BlockSpec((_HIDDEN, _HIDDEN), im_w) for _ in range(_NBLK)],
        pl.BlockSpec((_NBLK, _HIDDEN), im_w),
        pl.BlockSpec((_NBLK, _HIDDEN), im_w),
        pl.BlockSpec((_NBLK, _HIDDEN), im_w),
        pl.BlockSpec((_HIDDEN, _INPUT_DIM), im_w),
        pl.BlockSpec((1, _INPUT_DIM), im_w),
    ]
    return pl.pallas_call(
        _dec_body,
        grid=(_B // _BTD,),
        in_specs=in_specs,
        out_specs=pl.BlockSpec((_BTD, _INPUT_DIM), im_b),
        out_shape=jax.ShapeDtypeStruct((_B, _INPUT_DIM), _f32),
    )(z_q, z, dec['W0'], v0, *wb, bb, gb, beb, dec['Wf'], bf)


def kernel(x, params):
    enc, dec, emb = params['enc'], params['dec'], params['emb']
    z = _enc_mlp(enc, x)
    zn = jnp.linalg.norm(z, axis=1, keepdims=True)
    en = jnp.linalg.norm(emb, axis=1).reshape(1, _K)
    idx = _run_quant(z, zn, en, emb)
    z_q = _sc_gather(emb, idx)
    x_recon = _run_dec(z_q, z, dec)
    return (x_recon, z, z_q, idx)


# speedup vs baseline: 1.0824x; 1.0824x over previous
import functools

import jax
import jax.numpy as jnp
from jax import lax
from jax.experimental import pallas as pl
from jax.experimental.pallas import tpu as pltpu
from jax.experimental.pallas import tpu_sc as plsc

_INPUT_DIM = 1024
_HIDDEN = 512
_ENC_DIM = 256
_K = 8192
_NBLK = 6
_B = 4096
_BT = 1024
_BTD = 1024
_KC = 8192
_NW = 32
_BPW = _B // _NW

_f32 = jnp.float32


def _ln(x, g, b):
    m = jnp.mean(x, axis=-1, keepdims=True)
    v = jnp.var(x, axis=-1, keepdims=True)
    return (x - m) / jnp.sqrt(v + 1e-5) * g + b


def _enc_mlp(p, x):
    h = _ln(x @ p['W0'] + p['b0'], p['g0'], p['be0'])
    for blk in p['blocks']:
        t = jnp.tanh(_ln(h @ blk['W'] + blk['b'], blk['g'], blk['be']))
        t = _ln(t @ blk['W'] + blk['b'], blk['g'], blk['be'])
        h = jnp.tanh(t + h)
    return h @ p['Wf'] + p['bf']


def _quant_body(zf_ref, zn_ref, en_ref, e_ref, idx_ref):
    zf = zf_ref[...]
    zn = zn_ref[...]
    best_d = jnp.full((_BT, 1), 3.0e38, _f32)
    best_i = jnp.zeros((_BT, 1), jnp.int32)
    for c in range(_K // _KC):
        ec = e_ref[c * _KC:(c + 1) * _KC, :]
        en = en_ref[:, c * _KC:(c + 1) * _KC]
        s = lax.dot_general(zf, ec, (((1,), (1,)), ((), ())),
                            preferred_element_type=_f32)
        d = 1.0 - s / (zn * en)
        cm = jnp.min(d, axis=1, keepdims=True)
        io = lax.broadcasted_iota(jnp.int32, (_BT, _KC), 1)
        ci = jnp.min(jnp.where(d == cm, io, _K), axis=1, keepdims=True) + c * _KC
        upd = cm < best_d
        best_i = jnp.where(upd, ci, best_i)
        best_d = jnp.where(upd, cm, best_d)
    idx_ref[...] = jnp.broadcast_to(best_i, (_BT, 128))


def _run_quant(z, zn, en, emb):
    def im_b(i):
        return (i, 0)

    def im_w(i):
        return (0, 0)

    idx2d = pl.pallas_call(
        _quant_body,
        grid=(_B // _BT,),
        in_specs=[
            pl.BlockSpec((_BT, _ENC_DIM), im_b),
            pl.BlockSpec((_BT, 1), im_b),
            pl.BlockSpec((1, _K), im_w),
            pl.BlockSpec((_K, _ENC_DIM), im_w),
        ],
        out_specs=pl.BlockSpec((_BT, 128), im_b),
        out_shape=jax.ShapeDtypeStruct((_B, 128), jnp.int32),
    )(z, zn, en, emb)
    return idx2d[:, 0]


def _sc_gather(emb, idx):
    mesh = plsc.VectorSubcoreMesh(core_axis_name="c", subcore_axis_name="s")

    @functools.partial(
        pl.kernel,
        mesh=mesh,
        out_type=jax.ShapeDtypeStruct((_B, _ENC_DIM), _f32),
        scratch_types=[
            pltpu.VMEM((_BPW,), jnp.int32),
            pltpu.VMEM((_BPW, _ENC_DIM), _f32),
            pltpu.SemaphoreType.DMA,
        ],
    )
    def gk(table_hbm, idx_hbm, out_hbm, idx_v, rows_v, sem):
        wid = lax.axis_index("s") * 2 + lax.axis_index("c")
        base = wid * _BPW
        pltpu.sync_copy(idx_hbm.at[pl.ds(base, _BPW)], idx_v)
        pltpu.async_copy(table_hbm.at[idx_v], rows_v, sem).wait()
        pltpu.sync_copy(rows_v, out_hbm.at[pl.ds(base, _BPW)])

    return gk(emb, idx)


def _ln_in(h, g, b):
    m = jnp.mean(h, axis=-1, keepdims=True)
    v = jnp.mean((h - m) ** 2, axis=-1, keepdims=True)
    return (h - m) / jnp.sqrt(v + 1e-5) * g + b


def _dec_body(zq_ref, z_ref, w0_ref, v0_ref, wb0, wb1, wb2, wb3, wb4, wb5,
              bb_ref, gb_ref, beb_ref, wf_ref, bf_ref, out_ref):
    zz = z_ref[...]
    h_in = zz + (zq_ref[...] - zz)
    h = jnp.dot(h_in, w0_ref[...], preferred_element_type=_f32)
    h = _ln_in(h + v0_ref[0:1, :], v0_ref[1:2, :], v0_ref[2:3, :])
    w_refs = (wb0, wb1, wb2, wb3, wb4, wb5)
    for i in range(_NBLK):
        w = w_refs[i][...]
        b = bb_ref[i:i + 1, :]
        g = gb_ref[i:i + 1, :]
        be = beb_ref[i:i + 1, :]
        t = jnp.tanh(_ln_in(jnp.dot(h, w, preferred_element_type=_f32) + b, g, be))
        t = _ln_in(jnp.dot(t, w, preferred_element_type=_f32) + b, g, be)
        h = jnp.tanh(t + h)
    out_ref[...] = jnp.dot(h, wf_ref[...], preferred_element_type=_f32) + bf_ref[...]


def _run_dec(z_q, z, dec):
    wb = [blk['W'] for blk in dec['blocks']]
    bb = jnp.stack([blk['b'] for blk in dec['blocks']])
    gb = jnp.stack([blk['g'] for blk in dec['blocks']])
    beb = jnp.stack([blk['be'] for blk in dec['blocks']])
    v0 = jnp.stack([dec['b0'], dec['g0'], dec['be0']])
    bf = dec['bf'].reshape(1, _INPUT_DIM)

    def im_b(i):
        return (i, 0)

    def im_w(i):
        return (0, 0)

    in_specs = [
        pl.BlockSpec((_BTD, _ENC_DIM), im_b),
        pl.BlockSpec((_BTD, _ENC_DIM), im_b),
        pl.BlockSpec((_ENC_DIM, _HIDDEN), im_w),
        pl.BlockSpec((3, _HIDDEN), im_w),
        *[pl.BlockSpec((_HIDDEN, _HIDDEN), im_w) for _ in range(_NBLK)],
        pl.BlockSpec((_NBLK, _HIDDEN), im_w),
        pl.BlockSpec((_NBLK, _HIDDEN), im_w),
        pl.BlockSpec((_NBLK, _HIDDEN), im_w),
        pl.BlockSpec((_HIDDEN, _INPUT_DIM), im_w),
        pl.BlockSpec((1, _INPUT_DIM), im_w),
    ]
    return pl.pallas_call(
        _dec_body,
        grid=(_B // _BTD,),
        in_specs=in_specs,
        out_specs=pl.BlockSpec((_BTD, _INPUT_DIM), im_b),
        out_shape=jax.ShapeDtypeStruct((_B, _INPUT_DIM), _f32),
    )(z_q, z, dec['W0'], v0, *wb, bb, gb, beb, dec['Wf'], bf)


def kernel(x, params):
    enc, dec, emb = params['enc'], params['dec'], params['emb']
    z = _enc_mlp(enc, x)
    zn = jnp.linalg.norm(z, axis=1, keepdims=True)
    en = jnp.linalg.norm(emb, axis=1).reshape(1, _K)
    idx = _run_quant(z, zn, en, emb)
    z_q = _sc_gather(emb, idx)
    x_recon = _run_dec(z_q, z, dec)
    return (x_recon, z, z_q, idx)
